# baseline (device time: 25309 ns/iter reference)
import jax
import jax.numpy as jnp
from jax import lax
from jax.experimental import pallas as pl
from jax.experimental.pallas import tpu as pltpu

W = 8
M_PER = 512
D = 512
EPS = 1e-6

ORDERINGS = (
    ((4, 3, 1), 0, 176),
    ((1, 4, 3), 176, 176),
    ((3, 1, 4), 352, 160),
)
NEIGHBOR_MASKS = (1, 3, 4)
NMSG = 7


def kernel(partial, gamma):
    m_tot = partial.shape[1]
    assert partial.shape == (1, W * M_PER, D), partial.shape
    x = partial.reshape(m_tot, D)
    gamma2d = gamma.reshape(1, D)

    def body(x_hbm, g_ref, out_ref, *scratch):
        bufs = [scratch[6 * k: 6 * k + 6] for k in range(3)]
        x_ref, ssem, rsem, copy_sem = scratch[18:22]

        my = lax.axis_index("i")

        in_copy = pltpu.make_async_copy(x_hbm, x_ref, copy_sem)
        in_copy.start()

        barrier_sem = pltpu.get_barrier_semaphore()
        for m in NEIGHBOR_MASKS:
            pl.semaphore_signal(
                barrier_sem, inc=1,
                device_id=(my ^ m,), device_id_type=pl.DeviceIdType.MESH,
            )

        def stripe(c, off, h):
            return x_ref[pl.ds(c * M_PER + off, h), :]

        def mk(k, msg, src, dst, partner):
            return pltpu.make_async_remote_copy(
                src_ref=src, dst_ref=dst,
                send_sem=ssem.at[k, msg], recv_sem=rsem.at[k, msg],
                device_id=(partner,), device_id_type=pl.DeviceIdType.MESH,
            )

        spans = [(0, m3, m2, m2 ^ m3) for ((m1, m2, m3), _, _) in ORDERINGS]
        r1 = {}
        r2a, r2b, r3 = {}, {}, {}

        in_copy.wait()
        for k, ((m1, m2, m3), off, h) in enumerate(ORDERINGS):
            send1 = bufs[k][0]
            send1[3] = stripe(my ^ m1 ^ spans[k][3], off, h).astype(
                jnp.bfloat16)
        pl.semaphore_wait(barrier_sem, len(NEIGHBOR_MASKS))
        for j in (3, 2, 1, 0):
            for k, ((m1, m2, m3), off, h) in enumerate(ORDERINGS):
                send1 = bufs[k][0]
                if j != 3:
                    send1[j] = stripe(my ^ m1 ^ spans[k][j], off, h).astype(
                        jnp.bfloat16)
                r1[k, j] = mk(k, 3 - j, send1.at[j], bufs[k][1].at[j],
                              my ^ m1)
                r1[k, j].start()

        for j in (3, 2, 1, 0):
            for k, ((m1, m2, m3), off, h) in enumerate(ORDERINGS):
                bufs[k][2][j] = stripe(my ^ spans[k][j], off, h).astype(
                    jnp.bfloat16)

        for k, ((m1, m2, m3), off, h) in enumerate(ORDERINGS):
            send1, recv1, xk, acc, recv2, recv3 = bufs[k]
            r1[k, 3].wait()
            acc[3] = recv1[3] + xk[3]
            r2a[k] = mk(k, 4, acc.at[pl.ds(3, 1)], recv2.at[pl.ds(1, 1)],
                        my ^ m2)
            r2a[k].start()
        for k, ((m1, m2, m3), off, h) in enumerate(ORDERINGS):
            send1, recv1, xk, acc, recv2, recv3 = bufs[k]
            r1[k, 2].wait()
            acc[2] = recv1[2] + xk[2]
            r2b[k] = mk(k, 5, acc.at[pl.ds(2, 1)], recv2.at[pl.ds(0, 1)],
                        my ^ m2)
            r2b[k].start()

        for k in range(3):
            send1, recv1, xk, acc, recv2, recv3 = bufs[k]
            r1[k, 1].wait()
            acc[1] = recv1[1] + xk[1]

        for k, ((m1, m2, m3), off, h) in enumerate(ORDERINGS):
            send1, recv1, xk, acc, recv2, recv3 = bufs[k]
            r2a[k].wait()
            acc[1] = acc[1] + recv2[1]
            r3[k] = mk(k, 6, acc.at[pl.ds(1, 1)], recv3, my ^ m3)
            r3[k].start()

        for k in range(3):
            send1, recv1, xk, acc, recv2, recv3 = bufs[k]
            r1[k, 0].wait()
            acc[0] = recv1[0] + xk[0]
        for k in range(3):
            send1, recv1, xk, acc, recv2, recv3 = bufs[k]
            r2b[k].wait()
            acc[0] = acc[0] + recv2[0]

        for k, ((m1, m2, m3), off, h) in enumerate(ORDERINGS):
            send1, recv1, xk, acc, recv2, recv3 = bufs[k]
            r3[k].wait()
            y = acc[0].astype(jnp.float32) + recv3[0].astype(jnp.float32)
            rms = jnp.sqrt(jnp.mean(y * y, axis=-1, keepdims=True) + EPS)
            out_ref[pl.ds(off, h), :] = y / rms * g_ref[...]

    scratch_shapes = []
    for (_, _, h) in ORDERINGS:
        scratch_shapes += [
            pltpu.VMEM((4, h, D), jnp.bfloat16),
            pltpu.VMEM((4, h, D), jnp.bfloat16),
            pltpu.VMEM((4, h, D), jnp.bfloat16),
            pltpu.VMEM((4, h, D), jnp.bfloat16),
            pltpu.VMEM((2, h, D), jnp.bfloat16),
            pltpu.VMEM((1, h, D), jnp.bfloat16),
        ]
    scratch_shapes += [
        pltpu.VMEM((W * M_PER, D), jnp.float32),
        pltpu.SemaphoreType.DMA((3, NMSG)),
        pltpu.SemaphoreType.DMA((3, NMSG)),
        pltpu.SemaphoreType.DMA,
    ]

    return pl.pallas_call(
        body,
        out_shape=jax.ShapeDtypeStruct((M_PER, D), jnp.float32),
        in_specs=[
            pl.BlockSpec(memory_space=pltpu.MemorySpace.HBM),
            pl.BlockSpec(memory_space=pltpu.VMEM),
        ],
        out_specs=pl.BlockSpec(memory_space=pltpu.VMEM),
        scratch_shapes=scratch_shapes,
        compiler_params=pltpu.CompilerParams(collective_id=0),
    )(x, gamma2d)


# device time: 24796 ns/iter; 1.0207x vs baseline; 1.0207x over previous
import jax
import jax.numpy as jnp
from jax import lax
from jax.experimental import pallas as pl
from jax.experimental.pallas import tpu as pltpu

W = 8
M_PER = 512
D = 512
EPS = 1e-6

ORDERINGS = (
    ((4, 3, 1), 0, 176),
    ((1, 4, 3), 176, 176),
    ((3, 1, 4), 352, 160),
)
NEIGHBOR_MASKS = (1, 3, 4)
NMSG = 7


def kernel(partial, gamma):
    m_tot = partial.shape[1]
    assert partial.shape == (1, W * M_PER, D), partial.shape
    x = partial.reshape(m_tot, D)
    gamma2d = gamma.reshape(1, D)

    def body(x_ref, g_ref, out_ref, *scratch):
        bufs = [scratch[6 * k: 6 * k + 6] for k in range(3)]
        ssem, rsem = scratch[18], scratch[19]

        my = lax.axis_index("i")

        barrier_sem = pltpu.get_barrier_semaphore()
        for m in NEIGHBOR_MASKS:
            pl.semaphore_signal(
                barrier_sem, inc=1,
                device_id=(my ^ m,), device_id_type=pl.DeviceIdType.MESH,
            )
        pl.semaphore_wait(barrier_sem, len(NEIGHBOR_MASKS))

        def stripe(c, off, h):
            return x_ref[pl.ds(c * M_PER + off, h), :]

        def mk(k, msg, src, dst, partner):
            return pltpu.make_async_remote_copy(
                src_ref=src, dst_ref=dst,
                send_sem=ssem.at[k, msg], recv_sem=rsem.at[k, msg],
                device_id=(partner,), device_id_type=pl.DeviceIdType.MESH,
            )

        spans = [(0, m3, m2, m2 ^ m3) for ((m1, m2, m3), _, _) in ORDERINGS]
        r1 = {}
        r2a, r2b, r3 = {}, {}, {}

        for j in (3, 2, 1, 0):
            for k, ((m1, m2, m3), off, h) in enumerate(ORDERINGS):
                send1 = bufs[k][0]
                send1[j] = stripe(my ^ m1 ^ spans[k][j], off, h).astype(
                    jnp.bfloat16)
                r1[k, j] = mk(k, 3 - j, send1.at[j], bufs[k][1].at[j],
                              my ^ m1)
                r1[k, j].start()

        for j in (3, 2, 1, 0):
            for k, ((m1, m2, m3), off, h) in enumerate(ORDERINGS):
                bufs[k][2][j] = stripe(my ^ spans[k][j], off, h).astype(
                    jnp.bfloat16)

        for k, ((m1, m2, m3), off, h) in enumerate(ORDERINGS):
            send1, recv1, xk, acc, recv2, recv3 = bufs[k]
            r1[k, 3].wait()
            acc[3] = recv1[3] + xk[3]
            r2a[k] = mk(k, 4, acc.at[pl.ds(3, 1)], recv2.at[pl.ds(1, 1)],
                        my ^ m2)
            r2a[k].start()
        for k, ((m1, m2, m3), off, h) in enumerate(ORDERINGS):
            send1, recv1, xk, acc, recv2, recv3 = bufs[k]
            r1[k, 2].wait()
            acc[2] = recv1[2] + xk[2]
            r2b[k] = mk(k, 5, acc.at[pl.ds(2, 1)], recv2.at[pl.ds(0, 1)],
                        my ^ m2)
            r2b[k].start()

        for k in range(3):
            send1, recv1, xk, acc, recv2, recv3 = bufs[k]
            r1[k, 1].wait()
            acc[1] = recv1[1] + xk[1]

        for k, ((m1, m2, m3), off, h) in enumerate(ORDERINGS):
            send1, recv1, xk, acc, recv2, recv3 = bufs[k]
            r2a[k].wait()
            acc[1] = acc[1] + recv2[1]
            r3[k] = mk(k, 6, acc.at[pl.ds(1, 1)], recv3, my ^ m3)
            r3[k].start()

        for k in range(3):
            send1, recv1, xk, acc, recv2, recv3 = bufs[k]
            r1[k, 0].wait()
            acc[0] = recv1[0] + xk[0]
        for k in range(3):
            send1, recv1, xk, acc, recv2, recv3 = bufs[k]
            r2b[k].wait()
            acc[0] = acc[0] + recv2[0]

        for k, ((m1, m2, m3), off, h) in enumerate(ORDERINGS):
            send1, recv1, xk, acc, recv2, recv3 = bufs[k]
            r3[k].wait()
            y = acc[0].astype(jnp.float32) + recv3[0].astype(jnp.float32)
            rms = jnp.sqrt(jnp.mean(y * y, axis=-1, keepdims=True) + EPS)
            out_ref[pl.ds(off, h), :] = y / rms * g_ref[...]

    scratch_shapes = []
    for (_, _, h) in ORDERINGS:
        scratch_shapes += [
            pltpu.VMEM((4, h, D), jnp.bfloat16),
            pltpu.VMEM((4, h, D), jnp.bfloat16),
            pltpu.VMEM((4, h, D), jnp.bfloat16),
            pltpu.VMEM((4, h, D), jnp.bfloat16),
            pltpu.VMEM((2, h, D), jnp.bfloat16),
            pltpu.VMEM((1, h, D), jnp.bfloat16),
        ]
    scratch_shapes += [
        pltpu.SemaphoreType.DMA((3, NMSG)),
        pltpu.SemaphoreType.DMA((3, NMSG)),
    ]

    return pl.pallas_call(
        body,
        out_shape=jax.ShapeDtypeStruct((M_PER, D), jnp.float32),
        in_specs=[
            pl.BlockSpec(memory_space=pltpu.VMEM),
            pl.BlockSpec(memory_space=pltpu.VMEM),
        ],
        out_specs=pl.BlockSpec(memory_space=pltpu.VMEM),
        scratch_shapes=scratch_shapes,
        compiler_params=pltpu.CompilerParams(collective_id=0),
    )(x, gamma2d)


# device time: 24475 ns/iter; 1.0341x vs baseline; 1.0131x over previous
import jax
import jax.numpy as jnp
from jax import lax
from jax.experimental import pallas as pl
from jax.experimental.pallas import tpu as pltpu

W = 8
M_PER = 512
D = 512
EPS = 1e-6

ORDERINGS = (
    ((4, 3, 1), 0, 176),
    ((1, 4, 3), 176, 176),
    ((3, 1, 4), 352, 160),
)
NEIGHBOR_MASKS = (1, 3, 4)
CL = D // 2
NMSG = 10


def kernel(partial, gamma):
    m_tot = partial.shape[1]
    assert partial.shape == (1, W * M_PER, D), partial.shape
    x = partial.reshape(m_tot, D)
    gamma2d = gamma.reshape(1, D)

    def body(x_ref, g_ref, out_ref, *scratch):
        bufs = [scratch[6 * k: 6 * k + 6] for k in range(3)]
        ssem, rsem = scratch[18], scratch[19]

        my = lax.axis_index("i")

        barrier_sem = pltpu.get_barrier_semaphore()
        for m in NEIGHBOR_MASKS:
            pl.semaphore_signal(
                barrier_sem, inc=1,
                device_id=(my ^ m,), device_id_type=pl.DeviceIdType.MESH,
            )
        pl.semaphore_wait(barrier_sem, len(NEIGHBOR_MASKS))

        def stripe(c, off, h):
            return x_ref[pl.ds(c * M_PER + off, h), :]

        def mk(k, msg, src, dst, partner):
            return pltpu.make_async_remote_copy(
                src_ref=src, dst_ref=dst,
                send_sem=ssem.at[k, msg], recv_sem=rsem.at[k, msg],
                device_id=(partner,), device_id_type=pl.DeviceIdType.MESH,
            )

        spans = [(0, m3, m2, m2 ^ m3) for ((m1, m2, m3), _, _) in ORDERINGS]
        r1s3a, r1s3b, r1 = {}, {}, {}
        r2aa, r2ab, r2b, r3a, r3b = {}, {}, {}, {}, {}
        lo, hi = pl.ds(0, CL), pl.ds(CL, CL)

        for k, ((m1, m2, m3), off, h) in enumerate(ORDERINGS):
            send1, recv1 = bufs[k][0], bufs[k][1]
            send1[3] = stripe(my ^ m1 ^ spans[k][3], off, h).astype(
                jnp.bfloat16)
            r1s3a[k] = mk(k, 0, send1.at[3, :, lo], recv1.at[3, :, lo],
                          my ^ m1)
            r1s3a[k].start()
            r1s3b[k] = mk(k, 1, send1.at[3, :, hi], recv1.at[3, :, hi],
                          my ^ m1)
            r1s3b[k].start()
        for j in (2, 1, 0):
            for k, ((m1, m2, m3), off, h) in enumerate(ORDERINGS):
                send1, recv1 = bufs[k][0], bufs[k][1]
                send1[j] = stripe(my ^ m1 ^ spans[k][j], off, h).astype(
                    jnp.bfloat16)
                r1[k, j] = mk(k, 4 - j, send1.at[j], recv1.at[j], my ^ m1)
                r1[k, j].start()

        for j in (3, 2, 1, 0):
            for k, ((m1, m2, m3), off, h) in enumerate(ORDERINGS):
                bufs[k][2][j] = stripe(my ^ spans[k][j], off, h).astype(
                    jnp.bfloat16)

        for k, ((m1, m2, m3), off, h) in enumerate(ORDERINGS):
            send1, recv1, xk, acc, recv2, recv3 = bufs[k]
            r1s3a[k].wait()
            acc[3, :, lo] = recv1[3, :, lo] + xk[3, :, lo]
            r2aa[k] = mk(k, 5, acc.at[3, :, lo], recv2.at[1, :, lo],
                         my ^ m2)
            r2aa[k].start()
        for k, ((m1, m2, m3), off, h) in enumerate(ORDERINGS):
            send1, recv1, xk, acc, recv2, recv3 = bufs[k]
            r1s3b[k].wait()
            acc[3, :, hi] = recv1[3, :, hi] + xk[3, :, hi]
            r2ab[k] = mk(k, 6, acc.at[3, :, hi], recv2.at[1, :, hi],
                         my ^ m2)
            r2ab[k].start()
        for k, ((m1, m2, m3), off, h) in enumerate(ORDERINGS):
            send1, recv1, xk, acc, recv2, recv3 = bufs[k]
            r1[k, 2].wait()
            acc[2] = recv1[2] + xk[2]
            r2b[k] = mk(k, 7, acc.at[pl.ds(2, 1)], recv2.at[pl.ds(0, 1)],
                        my ^ m2)
            r2b[k].start()

        for k, ((m1, m2, m3), off, h) in enumerate(ORDERINGS):
            send1, recv1, xk, acc, recv2, recv3 = bufs[k]
            r1[k, 1].wait()
            r2aa[k].wait()
            acc[1, :, lo] = (recv1[1, :, lo] + xk[1, :, lo]
                             + recv2[1, :, lo])
            r3a[k] = mk(k, 8, acc.at[1, :, lo], recv3.at[0, :, lo],
                        my ^ m3)
            r3a[k].start()
        for k, ((m1, m2, m3), off, h) in enumerate(ORDERINGS):
            send1, recv1, xk, acc, recv2, recv3 = bufs[k]
            r2ab[k].wait()
            acc[1, :, hi] = (recv1[1, :, hi] + xk[1, :, hi]
                             + recv2[1, :, hi])
            r3b[k] = mk(k, 9, acc.at[1, :, hi], recv3.at[0, :, hi],
                        my ^ m3)
            r3b[k].start()

        for k in range(3):
            send1, recv1, xk, acc, recv2, recv3 = bufs[k]
            r1[k, 0].wait()
            acc[0] = recv1[0] + xk[0]
        for k in range(3):
            send1, recv1, xk, acc, recv2, recv3 = bufs[k]
            r2b[k].wait()
            acc[0] = acc[0] + recv2[0]

        for k, ((m1, m2, m3), off, h) in enumerate(ORDERINGS):
            send1, recv1, xk, acc, recv2, recv3 = bufs[k]
            r3a[k].wait()
            r3b[k].wait()
            y = acc[0].astype(jnp.float32) + recv3[0].astype(jnp.float32)
            rms = jnp.sqrt(jnp.mean(y * y, axis=-1, keepdims=True) + EPS)
            out_ref[pl.ds(off, h), :] = y / rms * g_ref[...]

    scratch_shapes = []
    for (_, _, h) in ORDERINGS:
        scratch_shapes += [
            pltpu.VMEM((4, h, D), jnp.bfloat16),
            pltpu.VMEM((4, h, D), jnp.bfloat16),
            pltpu.VMEM((4, h, D), jnp.bfloat16),
            pltpu.VMEM((4, h, D), jnp.bfloat16),
            pltpu.VMEM((2, h, D), jnp.bfloat16),
            pltpu.VMEM((1, h, D), jnp.bfloat16),
        ]
    scratch_shapes += [
        pltpu.SemaphoreType.DMA((3, NMSG)),
        pltpu.SemaphoreType.DMA((3, NMSG)),
    ]

    return pl.pallas_call(
        body,
        out_shape=jax.ShapeDtypeStruct((M_PER, D), jnp.float32),
        in_specs=[
            pl.BlockSpec(memory_space=pltpu.VMEM),
            pl.BlockSpec(memory_space=pltpu.VMEM),
        ],
        out_specs=pl.BlockSpec(memory_space=pltpu.VMEM),
        scratch_shapes=scratch_shapes,
        compiler_params=pltpu.CompilerParams(collective_id=0),
    )(x, gamma2d)
